# Initial kernel scaffold; baseline (speedup 1.0000x reference)
import jax
import jax.numpy as jnp
from jax import lax
from jax.experimental import pallas as pl
from jax.experimental.pallas import tpu as pltpu
from jax.experimental.pallas import tpu_sc as plsc

D = 64
SEQ = 200
BATCH = 4096
NC = 2
NS = 16
NW = NC * NS
SEQ_PER_W = BATCH // NW
EPS = 1e-12


def _rsqrt(x):
    i = lax.bitcast_convert_type(x, jnp.int32)
    i = jnp.int32(0x5F3759DF) - (i >> 1)
    y = lax.bitcast_convert_type(i, jnp.float32)
    for _ in range(3):
        y = y * (1.5 - 0.5 * x * y * y)
    return y


def _body(x_hbm, w_hbm, p_hbm, g_hbm, b_hbm, out_hbm,
          p_v, g_v, b_v, idx_v, rows_v, out_v, sem):
    wid = lax.axis_index("s") * NC + lax.axis_index("c")

    pltpu.sync_copy(p_hbm, p_v)
    pltpu.sync_copy(g_hbm, g_v)
    pltpu.sync_copy(b_hbm, b_v)

    g0 = g_v[pl.ds(0, 16)]
    g1 = g_v[pl.ds(16, 16)]
    g2 = g_v[pl.ds(32, 16)]
    g3 = g_v[pl.ds(48, 16)]
    bb0 = b_v[pl.ds(0, 16)]
    bb1 = b_v[pl.ds(16, 16)]
    bb2 = b_v[pl.ds(32, 16)]
    bb3 = b_v[pl.ds(48, 16)]

    def seq_body(s, _):
        seq = wid * SEQ_PER_W + s
        pltpu.sync_copy(x_hbm.at[seq], idx_v)
        h0 = pltpu.async_copy(w_hbm.at[idx_v.at[0]], rows_v.at[pl.ds(0, 100)], sem)
        h1 = pltpu.async_copy(w_hbm.at[idx_v.at[1]], rows_v.at[pl.ds(100, 100)], sem)
        h0.wait()
        h1.wait()

        def row_body(r, _):
            e0 = rows_v[r, pl.ds(0, 16)] + p_v[r, pl.ds(0, 16)]
            e1 = rows_v[r, pl.ds(16, 16)] + p_v[r, pl.ds(16, 16)]
            e2 = rows_v[r, pl.ds(32, 16)] + p_v[r, pl.ds(32, 16)]
            e3 = rows_v[r, pl.ds(48, 16)] + p_v[r, pl.ds(48, 16)]
            tot = jnp.sum(e0 + e1 + e2 + e3)
            tot2 = jnp.sum(e0 * e0 + e1 * e1 + e2 * e2 + e3 * e3)
            mean = tot * (1.0 / D)
            var = tot2 * (1.0 / D) - mean * mean
            inv = _rsqrt(var + EPS)
            shift = -mean * inv
            out_v[r, pl.ds(0, 16)] = (e0 * inv + shift) * g0 + bb0
            out_v[r, pl.ds(16, 16)] = (e1 * inv + shift) * g1 + bb1
            out_v[r, pl.ds(32, 16)] = (e2 * inv + shift) * g2 + bb2
            out_v[r, pl.ds(48, 16)] = (e3 * inv + shift) * g3 + bb3
            return 0

        lax.fori_loop(0, SEQ, row_body, 0)
        pltpu.sync_copy(out_v, out_hbm.at[seq])
        return 0

    lax.fori_loop(0, SEQ_PER_W, seq_body, 0)


def kernel(x, W, P, gamma, beta):
    x2 = x.reshape(BATCH, 2, SEQ // 2).astype(jnp.int32)
    mesh = plsc.VectorSubcoreMesh(core_axis_name="c", subcore_axis_name="s")
    run = pl.kernel(
        _body,
        out_type=jax.ShapeDtypeStruct((BATCH, SEQ, D), jnp.float32),
        mesh=mesh,
        scratch_types=[
            pltpu.VMEM((SEQ, D), jnp.float32),
            pltpu.VMEM((D,), jnp.float32),
            pltpu.VMEM((D,), jnp.float32),
            pltpu.VMEM((2, SEQ // 2), jnp.int32),
            pltpu.VMEM((SEQ, D), jnp.float32),
            pltpu.VMEM((SEQ, D), jnp.float32),
            pltpu.SemaphoreType.DMA,
        ],
    )
    return run(x2, W, P, gamma, beta)


# speedup vs baseline: 2.1778x; 2.1778x over previous
import jax
import jax.numpy as jnp
from jax import lax
from jax.experimental import pallas as pl
from jax.experimental.pallas import tpu as pltpu
from jax.experimental.pallas import tpu_sc as plsc

D = 64
SEQ = 200
BATCH = 4096
NC = 2
NS = 16
NW = NC * NS
SEQ_PER_W = BATCH // NW
EPS = 1e-12


def _lane_sum(v, perms):
    dn = lax.GatherDimensionNumbers(
        offset_dims=(), collapsed_slice_dims=(0,), start_index_map=(0,))
    for p in perms:
        v = v + lax.gather(v, p[:, None], dimension_numbers=dn,
                           slice_sizes=(1,),
                           mode=lax.GatherScatterMode.PROMISE_IN_BOUNDS)
    return v


def _rsqrt(x):
    i = lax.bitcast_convert_type(x, jnp.int32)
    i = jnp.int32(0x5F3759DF) - (i >> 1)
    y = lax.bitcast_convert_type(i, jnp.float32)
    for _ in range(3):
        y = y * (1.5 - 0.5 * x * y * y)
    return y


def _body(x_hbm, w_hbm, p_hbm, g_hbm, b_hbm, out_hbm,
          p_v, g_v, b_v, idx_v, rows_v, out_v, sem):
    wid = lax.axis_index("s") * NC + lax.axis_index("c")

    pltpu.sync_copy(p_hbm, p_v)
    pltpu.sync_copy(g_hbm, g_v)
    pltpu.sync_copy(b_hbm, b_v)

    g0 = g_v[pl.ds(0, 16)]
    g1 = g_v[pl.ds(16, 16)]
    g2 = g_v[pl.ds(32, 16)]
    g3 = g_v[pl.ds(48, 16)]
    bb0 = b_v[pl.ds(0, 16)]
    bb1 = b_v[pl.ds(16, 16)]
    bb2 = b_v[pl.ds(32, 16)]
    bb3 = b_v[pl.ds(48, 16)]

    iota = lax.iota(jnp.int32, 16)
    perms = [iota ^ k for k in (1, 2, 4, 8)]

    def seq_body(s, _):
        seq = wid * SEQ_PER_W + s
        pltpu.sync_copy(x_hbm.at[seq], idx_v)
        h0 = pltpu.async_copy(w_hbm.at[idx_v.at[0]], rows_v.at[pl.ds(0, 100)], sem)
        h1 = pltpu.async_copy(w_hbm.at[idx_v.at[1]], rows_v.at[pl.ds(100, 100)], sem)
        h0.wait()
        h1.wait()

        def row_body(r, _):
            e0 = rows_v[r, pl.ds(0, 16)] + p_v[r, pl.ds(0, 16)]
            e1 = rows_v[r, pl.ds(16, 16)] + p_v[r, pl.ds(16, 16)]
            e2 = rows_v[r, pl.ds(32, 16)] + p_v[r, pl.ds(32, 16)]
            e3 = rows_v[r, pl.ds(48, 16)] + p_v[r, pl.ds(48, 16)]
            tot = _lane_sum(e0 + e1 + e2 + e3, perms)
            tot2 = _lane_sum(e0 * e0 + e1 * e1 + e2 * e2 + e3 * e3, perms)
            mean = tot * (1.0 / D)
            var = tot2 * (1.0 / D) - mean * mean
            inv = _rsqrt(var + EPS)
            shift = -mean * inv
            out_v[r, pl.ds(0, 16)] = (e0 * inv + shift) * g0 + bb0
            out_v[r, pl.ds(16, 16)] = (e1 * inv + shift) * g1 + bb1
            out_v[r, pl.ds(32, 16)] = (e2 * inv + shift) * g2 + bb2
            out_v[r, pl.ds(48, 16)] = (e3 * inv + shift) * g3 + bb3
            return 0

        lax.fori_loop(0, SEQ, row_body, 0)
        pltpu.sync_copy(out_v, out_hbm.at[seq])
        return 0

    lax.fori_loop(0, SEQ_PER_W, seq_body, 0)


def kernel(x, W, P, gamma, beta):
    x2 = x.reshape(BATCH, 2, SEQ // 2).astype(jnp.int32)
    mesh = plsc.VectorSubcoreMesh(core_axis_name="c", subcore_axis_name="s")
    run = pl.kernel(
        _body,
        out_type=jax.ShapeDtypeStruct((BATCH, SEQ, D), jnp.float32),
        mesh=mesh,
        compiler_params=pltpu.CompilerParams(use_tc_tiling_on_sc=False),
        scratch_types=[
            pltpu.VMEM((SEQ, D), jnp.float32),
            pltpu.VMEM((D,), jnp.float32),
            pltpu.VMEM((D,), jnp.float32),
            pltpu.VMEM((2, SEQ // 2), jnp.int32),
            pltpu.VMEM((SEQ, D), jnp.float32),
            pltpu.VMEM((SEQ, D), jnp.float32),
            pltpu.SemaphoreType.DMA,
        ],
    )
    return run(x2, W, P, gamma, beta)


# speedup vs baseline: 2.4934x; 1.1449x over previous
import jax
import jax.numpy as jnp
from jax import lax
from jax.experimental import pallas as pl
from jax.experimental.pallas import tpu as pltpu
from jax.experimental.pallas import tpu_sc as plsc

D = 64
SEQ = 200
BATCH = 4096
NC = 2
NS = 16
NW = NC * NS
SEQ_PER_W = BATCH // NW
EPS = 1e-12


def _lane_sum(v, perms):
    dn = lax.GatherDimensionNumbers(
        offset_dims=(), collapsed_slice_dims=(0,), start_index_map=(0,))
    for p in perms:
        v = v + lax.gather(v, p[:, None], dimension_numbers=dn,
                           slice_sizes=(1,),
                           mode=lax.GatherScatterMode.PROMISE_IN_BOUNDS)
    return v


def _rsqrt(x):
    i = lax.bitcast_convert_type(x, jnp.int32)
    i = jnp.int32(0x5F3759DF) - (i >> 1)
    y = lax.bitcast_convert_type(i, jnp.float32)
    for _ in range(3):
        y = y * (1.5 - 0.5 * x * y * y)
    return y


def _body(x_hbm, w_hbm, p_hbm, g_hbm, b_hbm, out_hbm,
          p_v, g_v, b_v, idx_v, rows_v, out_v, sem_g, sem_o):
    wid = lax.axis_index("s") * NC + lax.axis_index("c")

    pltpu.sync_copy(p_hbm, p_v)
    pltpu.sync_copy(g_hbm, g_v)
    pltpu.sync_copy(b_hbm, b_v)

    g0 = g_v[pl.ds(0, 16)]
    g1 = g_v[pl.ds(16, 16)]
    g2 = g_v[pl.ds(32, 16)]
    g3 = g_v[pl.ds(48, 16)]
    bb0 = b_v[pl.ds(0, 16)]
    bb1 = b_v[pl.ds(16, 16)]
    bb2 = b_v[pl.ds(32, 16)]
    bb3 = b_v[pl.ds(48, 16)]

    iota = lax.iota(jnp.int32, 16)
    perms = [iota ^ k for k in (1, 2, 4, 8)]

    def start_gather(b, seq):
        pltpu.sync_copy(x_hbm.at[seq], idx_v.at[b])
        pltpu.async_copy(w_hbm.at[idx_v.at[b, 0]], rows_v.at[b, pl.ds(0, 100)],
                         sem_g.at[b])
        pltpu.async_copy(w_hbm.at[idx_v.at[b, 1]], rows_v.at[b, pl.ds(100, 100)],
                         sem_g.at[b])

    def wait_gather(b):
        pltpu.make_async_copy(w_hbm.at[idx_v.at[b, 0]],
                              rows_v.at[b, pl.ds(0, 100)], sem_g.at[b]).wait()
        pltpu.make_async_copy(w_hbm.at[idx_v.at[b, 1]],
                              rows_v.at[b, pl.ds(100, 100)], sem_g.at[b]).wait()

    def wait_out(b, seq):
        pltpu.make_async_copy(out_v.at[b], out_hbm.at[seq], sem_o.at[b]).wait()

    def compute(b):
        @plsc.parallel_loop(0, SEQ, unroll=4)
        def row_body(r):
            e0 = rows_v[b, r, pl.ds(0, 16)] + p_v[r, pl.ds(0, 16)]
            e1 = rows_v[b, r, pl.ds(16, 16)] + p_v[r, pl.ds(16, 16)]
            e2 = rows_v[b, r, pl.ds(32, 16)] + p_v[r, pl.ds(32, 16)]
            e3 = rows_v[b, r, pl.ds(48, 16)] + p_v[r, pl.ds(48, 16)]
            tot = _lane_sum(e0 + e1 + e2 + e3, perms)
            tot2 = _lane_sum(e0 * e0 + e1 * e1 + e2 * e2 + e3 * e3, perms)
            mean = tot * (1.0 / D)
            var = tot2 * (1.0 / D) - mean * mean
            inv = _rsqrt(var + EPS)
            shift = -mean * inv
            out_v[b, r, pl.ds(0, 16)] = (e0 * inv + shift) * g0 + bb0
            out_v[b, r, pl.ds(16, 16)] = (e1 * inv + shift) * g1 + bb1
            out_v[b, r, pl.ds(32, 16)] = (e2 * inv + shift) * g2 + bb2
            out_v[b, r, pl.ds(48, 16)] = (e3 * inv + shift) * g3 + bb3

    seq0 = wid * SEQ_PER_W
    start_gather(0, seq0)

    def outer(i, _):
        for b in (0, 1):
            s = 2 * i + b
            seq = seq0 + s
            start_gather(1 - b, jnp.minimum(seq + 1, BATCH - 1))
            wait_gather(b)

            @pl.when(i >= 1)
            def _():
                wait_out(b, jnp.maximum(seq - 2, 0))

            compute(b)
            pltpu.async_copy(out_v.at[b], out_hbm.at[seq], sem_o.at[b])
        return 0

    lax.fori_loop(0, SEQ_PER_W // 2, outer, 0)
    wait_gather(0)
    wait_out(0, seq0 + SEQ_PER_W - 2)
    wait_out(1, seq0 + SEQ_PER_W - 1)


def kernel(x, W, P, gamma, beta):
    x2 = x.reshape(BATCH, 2, SEQ // 2).astype(jnp.int32)
    mesh = plsc.VectorSubcoreMesh(core_axis_name="c", subcore_axis_name="s")
    run = pl.kernel(
        _body,
        out_type=jax.ShapeDtypeStruct((BATCH, SEQ, D), jnp.float32),
        mesh=mesh,
        compiler_params=pltpu.CompilerParams(use_tc_tiling_on_sc=False),
        scratch_types=[
            pltpu.VMEM((SEQ, D), jnp.float32),
            pltpu.VMEM((D,), jnp.float32),
            pltpu.VMEM((D,), jnp.float32),
            pltpu.VMEM((2, 2, SEQ // 2), jnp.int32),
            pltpu.VMEM((2, SEQ, D), jnp.float32),
            pltpu.VMEM((2, SEQ, D), jnp.float32),
            pltpu.SemaphoreType.DMA((2,)),
            pltpu.SemaphoreType.DMA((2,)),
        ],
    )
    return run(x2, W, P, gamma, beta)
